# Initial kernel scaffold; baseline (speedup 1.0000x reference)
import jax
import jax.numpy as jnp
from jax.experimental import pallas as pl


def kernel(latents, output_task_indices, output_values, output_weights, W, b):
    raise NotImplementedError("write your pallas kernel here")


# speedup vs baseline: 3.3214x; 3.3214x over previous
import jax
import jax.numpy as jnp
from jax import lax
from jax.experimental import pallas as pl
from jax.experimental.pallas import tpu as pltpu
from jax.experimental.pallas import tpu_sc as plsc

_B, _N, _D, _C = 16, 4096, 1024, 8
_T = 32768

_NC, _NS, _L = 2, 16, 16
_NW = _NC * _NS
_TPW = _T // _NW
_GCHUNK = 128
_NCHUNK = _TPW // _GCHUNK
_NGRP = _TPW // _L


def _proj_body(l_ref, w_ref, b_ref, p_ref):
    p_ref[...] = (
        jnp.dot(l_ref[...], w_ref[...], preferred_element_type=jnp.float32)
        + b_ref[...]
    )


def _project(latents_flat, W, b2):
    M = latents_flat.shape[0]
    BM = 4096
    return pl.pallas_call(
        _proj_body,
        grid=(M // BM,),
        in_specs=[
            pl.BlockSpec((BM, _D), lambda i: (i, 0)),
            pl.BlockSpec((_D, _C), lambda i: (0, 0)),
            pl.BlockSpec((1, _C), lambda i: (0, 0)),
        ],
        out_specs=pl.BlockSpec((BM, _C), lambda i: (i, 0)),
        out_shape=jax.ShapeDtypeStruct((M, _C), jnp.float32),
    )(latents_flat, W, b2)


def _sc_body(p_hbm, idx_hbm, vals_hbm, w_hbm,
             out_hbm, loss_hbm, pres_hbm,
             idx_v, gidx_v, rows_v, vals_v, w_v,
             loss_v, pres_v, sem):
    cid = lax.axis_index("c")
    sid = lax.axis_index("s")
    wid = sid * _NC + cid
    base = wid * _TPW

    pltpu.sync_copy(idx_hbm.at[pl.ds(base, _TPW)], idx_v)
    pltpu.sync_copy(vals_hbm.at[pl.ds(base, _TPW)], vals_v)
    pltpu.sync_copy(w_hbm.at[pl.ds(base, _TPW)], w_v)

    pres_v[...] = jnp.zeros((_L,), jnp.int32)
    ones = jnp.ones((_L,), jnp.int32)
    col0 = jnp.zeros((_L,), jnp.int32)
    col1 = jnp.ones((_L,), jnp.int32)

    for k in range(_NGRP):
        ids = lax.iota(jnp.int32, _L) + k * _L
        vb = plsc.load_gather(idx_v, [ids, col0])
        vn = plsc.load_gather(idx_v, [ids, col1])
        g = vb * _N + vn
        gpc = _GCHUNK // _L
        gidx_v[k // gpc, pl.ds((k % gpc) * _L, _L)] = g
        plsc.store_scatter(pres_v, [vb], ones)

    copies = [
        pltpu.async_copy(
            p_hbm.at[gidx_v.at[j]],
            rows_v.at[pl.ds(j * _GCHUNK, _GCHUNK)],
            sem,
        )
        for j in range(_NCHUNK)
    ]
    for cp in copies:
        cp.wait()

    pltpu.sync_copy(rows_v, out_hbm.at[pl.ds(base, _TPW)])

    acc = jnp.zeros((_L,), jnp.float32)
    for k in range(_NGRP):
        ids = lax.iota(jnp.int32, _L) + k * _L
        s = jnp.zeros((_L,), jnp.float32)
        for c in range(_C):
            cc = jnp.full((_L,), c, jnp.int32)
            rv = plsc.load_gather(rows_v, [ids, cc])
            vv = plsc.load_gather(vals_v, [ids, cc])
            d = rv - vv
            s = s + d * d
        acc = acc + w_v[pl.ds(k * _L, _L)] * s
    loss_v[...] = acc
    pltpu.sync_copy(loss_v, loss_hbm.at[wid])
    pltpu.sync_copy(pres_v, pres_hbm.at[wid])


def _sc_call(P, output_task_indices, output_values, output_weights):
    mesh = plsc.VectorSubcoreMesh(
        core_axis_name="c", subcore_axis_name="s",
        num_cores=_NC, num_subcores=_NS,
    )
    f = pl.kernel(
        _sc_body,
        out_type=[
            jax.ShapeDtypeStruct((_T, _C), jnp.float32),
            jax.ShapeDtypeStruct((_NW, _L), jnp.float32),
            jax.ShapeDtypeStruct((_NW, _L), jnp.int32),
        ],
        mesh=mesh,
        compiler_params=pltpu.CompilerParams(
            needs_layout_passes=False, use_tc_tiling_on_sc=False),
        scratch_types=[
            pltpu.VMEM((_TPW, 2), jnp.int32),
            pltpu.VMEM((_NCHUNK, _GCHUNK), jnp.int32),
            pltpu.VMEM((_TPW, _C), jnp.float32),
            pltpu.VMEM((_TPW, _C), jnp.float32),
            pltpu.VMEM((_TPW,), jnp.float32),
            pltpu.VMEM((_L,), jnp.float32),
            pltpu.VMEM((_L,), jnp.int32),
            pltpu.SemaphoreType.DMA,
        ],
    )
    return f(P, output_task_indices, output_values, output_weights)


def kernel(latents, output_task_indices, output_values, output_weights, W, b):
    latents_flat = latents.reshape(_B * _N, _D)
    P = _project(latents_flat, W, b.reshape(1, _C))
    out, loss_parts, pres = _sc_call(
        P, output_task_indices, output_values, output_weights)
    total = jnp.sum(loss_parts)
    nbatch = jnp.sum((jnp.sum(pres, axis=0) > 0).astype(jnp.float32))
    loss = total / (_T * _C) * nbatch / _B
    return out, loss


# speedup vs baseline: 4.9241x; 1.4825x over previous
import jax
import jax.numpy as jnp
from jax import lax
from jax.experimental import pallas as pl
from jax.experimental.pallas import tpu as pltpu
from jax.experimental.pallas import tpu_sc as plsc

_B, _N, _D, _C = 16, 4096, 1024, 8
_T = 32768

_NC, _NS, _L = 2, 16, 16
_NW = _NC * _NS
_TPW = _T // _NW
_GCHUNK = 128
_NBLK = _TPW // _GCHUNK
_GPB = _GCHUNK // _L


def _proj_body(l_ref, w_ref, b_ref, p_ref):
    p_ref[...] = (
        jnp.dot(l_ref[...], w_ref[...], preferred_element_type=jnp.float32)
        + b_ref[...]
    )


def _project(latents_flat, W, b2):
    M = latents_flat.shape[0]
    BM = 4096
    return pl.pallas_call(
        _proj_body,
        grid=(M // BM,),
        in_specs=[
            pl.BlockSpec((BM, _D), lambda i: (i, 0)),
            pl.BlockSpec((_D, _C), lambda i: (0, 0)),
            pl.BlockSpec((1, _C), lambda i: (0, 0)),
        ],
        out_specs=pl.BlockSpec((BM, _C), lambda i: (i, 0)),
        out_shape=jax.ShapeDtypeStruct((M, _C), jnp.float32),
    )(latents_flat, W, b2)


def _sc_body(p_hbm, idx3_hbm, vals3_hbm, w_hbm,
             out3_hbm, loss_hbm, pres_hbm,
             idx_v, vals_v, w_v, gidx_v, rows_v,
             out_v, loss_v, pres_v, sem):
    cid = lax.axis_index("c")
    sid = lax.axis_index("s")
    wid = sid * _NC + cid
    base = wid * _TPW
    t0 = wid * _NBLK

    pltpu.sync_copy(idx3_hbm.at[pl.ds(t0, _NBLK)], idx_v)
    pltpu.sync_copy(vals3_hbm.at[pl.ds(t0, _NBLK)], vals_v)
    pltpu.sync_copy(w_hbm.at[pl.ds(base, _TPW)], w_v)

    pres_v[...] = jnp.zeros((_L,), jnp.int32)
    ones = jnp.ones((_L,), jnp.int32)

    copies = []
    for tb in range(_NBLK):
        for g in range(_GPB):
            sl = pl.ds(g * _L, _L)
            vb = idx_v[tb, 0, sl]
            vn = idx_v[tb, 1, sl]
            gidx_v[tb, sl] = vb * _N + vn
            plsc.store_scatter(pres_v, [vb], ones)
        copies.append(pltpu.async_copy(
            p_hbm.at[gidx_v.at[tb]],
            rows_v.at[pl.ds(tb * _GCHUNK, _GCHUNK)],
            sem,
        ))

    acc = jnp.zeros((_L,), jnp.float32)
    for tb in range(_NBLK):
        copies[tb].wait()
        for g in range(_GPB):
            ids = lax.iota(jnp.int32, _L) + (tb * _GCHUNK + g * _L)
            sl = pl.ds(g * _L, _L)
            s = jnp.zeros((_L,), jnp.float32)
            for c in range(_C):
                cc = jnp.full((_L,), c, jnp.int32)
                rv = plsc.load_gather(rows_v, [ids, cc])
                out_v[c, sl] = rv
                d = rv - vals_v[tb, c, sl]
                s = s + d * d
            acc = acc + w_v[pl.ds(tb * _GCHUNK + g * _L, _L)] * s
        pltpu.sync_copy(out_v, out3_hbm.at[t0 + tb])

    loss_v[...] = acc
    pltpu.sync_copy(loss_v, loss_hbm.at[wid])
    pltpu.sync_copy(pres_v, pres_hbm.at[wid])


def _sc_call(P, idx3, vals3, output_weights):
    mesh = plsc.VectorSubcoreMesh(
        core_axis_name="c", subcore_axis_name="s",
        num_cores=_NC, num_subcores=_NS,
    )
    f = pl.kernel(
        _sc_body,
        out_type=[
            jax.ShapeDtypeStruct((_T // _GCHUNK, _C, _GCHUNK), jnp.float32),
            jax.ShapeDtypeStruct((_NW, _L), jnp.float32),
            jax.ShapeDtypeStruct((_NW, _L), jnp.int32),
        ],
        mesh=mesh,
        compiler_params=pltpu.CompilerParams(
            needs_layout_passes=False, use_tc_tiling_on_sc=False),
        scratch_types=[
            pltpu.VMEM((_NBLK, 2, _GCHUNK), jnp.int32),
            pltpu.VMEM((_NBLK, _C, _GCHUNK), jnp.float32),
            pltpu.VMEM((_TPW,), jnp.float32),
            pltpu.VMEM((_NBLK, _GCHUNK), jnp.int32),
            pltpu.VMEM((_TPW, _C), jnp.float32),
            pltpu.VMEM((_C, _GCHUNK), jnp.float32),
            pltpu.VMEM((_L,), jnp.float32),
            pltpu.VMEM((_L,), jnp.int32),
            pltpu.SemaphoreType.DMA,
        ],
    )
    return f(P, idx3, vals3, output_weights)


def kernel(latents, output_task_indices, output_values, output_weights, W, b):
    latents_flat = latents.reshape(_B * _N, _D)
    P = _project(latents_flat, W, b.reshape(1, _C))
    idx3 = output_task_indices.reshape(_T // 128, 128, 2).transpose(0, 2, 1)
    vals3 = output_values.reshape(_T // 128, 128, _C).transpose(0, 2, 1)
    out3, loss_parts, pres = _sc_call(P, idx3, vals3, output_weights)
    out = out3.transpose(0, 2, 1).reshape(_T, _C)
    total = jnp.sum(loss_parts)
    nbatch = jnp.sum((jnp.sum(pres, axis=0) > 0).astype(jnp.float32))
    loss = total / (_T * _C) * nbatch / _B
    return out, loss


# speedup vs baseline: 4.9337x; 1.0020x over previous
import jax
import jax.numpy as jnp
from jax import lax
from jax.experimental import pallas as pl
from jax.experimental.pallas import tpu as pltpu
from jax.experimental.pallas import tpu_sc as plsc

_B, _N, _D, _C = 16, 4096, 1024, 8
_T = 32768

_NC, _NS, _L = 2, 16, 16
_NW = _NC * _NS
_TPW = _T // _NW
_GCHUNK = 128
_NBLK = _TPW // _GCHUNK
_GPB = _GCHUNK // _L


def _proj_body(l_ref, w_ref, b_ref, p_ref):
    p_ref[...] = (
        jnp.dot(l_ref[...], w_ref[...], preferred_element_type=jnp.float32)
        + b_ref[...]
    )


def _project(latents_flat, W, b2):
    M = latents_flat.shape[0]
    BM = 2048
    return pl.pallas_call(
        _proj_body,
        grid=(M // BM,),
        in_specs=[
            pl.BlockSpec((BM, _D), lambda i: (i, 0)),
            pl.BlockSpec((_D, _C), lambda i: (0, 0)),
            pl.BlockSpec((1, _C), lambda i: (0, 0)),
        ],
        out_specs=pl.BlockSpec((BM, _C), lambda i: (i, 0)),
        out_shape=jax.ShapeDtypeStruct((M, _C), jnp.float32),
    )(latents_flat, W, b2)


def _sc_body(p_hbm, idx3_hbm, vals3_hbm, w_hbm,
             out3_hbm, loss_hbm, pres_hbm,
             idx_v, vals_v, w_v, gidx_v, rows_v,
             out_v, loss_v, pres_v, sem):
    cid = lax.axis_index("c")
    sid = lax.axis_index("s")
    wid = sid * _NC + cid
    base = wid * _TPW
    t0 = wid * _NBLK

    pltpu.sync_copy(idx3_hbm.at[pl.ds(t0, _NBLK)], idx_v)
    pltpu.sync_copy(vals3_hbm.at[pl.ds(t0, _NBLK)], vals_v)
    pltpu.sync_copy(w_hbm.at[pl.ds(base, _TPW)], w_v)

    pres_v[...] = jnp.zeros((_L,), jnp.int32)
    ones = jnp.ones((_L,), jnp.int32)

    copies = []
    for tb in range(_NBLK):
        for g in range(_GPB):
            sl = pl.ds(g * _L, _L)
            vb = idx_v[tb, 0, sl]
            vn = idx_v[tb, 1, sl]
            gidx_v[tb, sl] = vb * _N + vn
            plsc.store_scatter(pres_v, [vb], ones)
        copies.append(pltpu.async_copy(
            p_hbm.at[gidx_v.at[tb]],
            rows_v.at[pl.ds(tb * _GCHUNK, _GCHUNK)],
            sem,
        ))

    acc = jnp.zeros((_L,), jnp.float32)
    for tb in range(_NBLK):
        copies[tb].wait()
        for g in range(_GPB):
            ids = lax.iota(jnp.int32, _L) + (tb * _GCHUNK + g * _L)
            sl = pl.ds(g * _L, _L)
            s = jnp.zeros((_L,), jnp.float32)
            for c in range(_C):
                cc = jnp.full((_L,), c, jnp.int32)
                rv = plsc.load_gather(rows_v, [ids, cc])
                out_v[c, sl] = rv
                d = rv - vals_v[tb, c, sl]
                s = s + d * d
            acc = acc + w_v[pl.ds(tb * _GCHUNK + g * _L, _L)] * s
        pltpu.sync_copy(out_v, out3_hbm.at[t0 + tb])

    loss_v[...] = acc
    pltpu.sync_copy(loss_v, loss_hbm.at[wid])
    pltpu.sync_copy(pres_v, pres_hbm.at[wid])


def _sc_call(P, idx3, vals3, output_weights):
    mesh = plsc.VectorSubcoreMesh(
        core_axis_name="c", subcore_axis_name="s",
        num_cores=_NC, num_subcores=_NS,
    )
    f = pl.kernel(
        _sc_body,
        out_type=[
            jax.ShapeDtypeStruct((_T // _GCHUNK, _C, _GCHUNK), jnp.float32),
            jax.ShapeDtypeStruct((_NW, _L), jnp.float32),
            jax.ShapeDtypeStruct((_NW, _L), jnp.int32),
        ],
        mesh=mesh,
        compiler_params=pltpu.CompilerParams(
            needs_layout_passes=False, use_tc_tiling_on_sc=False),
        scratch_types=[
            pltpu.VMEM((_NBLK, 2, _GCHUNK), jnp.int32),
            pltpu.VMEM((_NBLK, _C, _GCHUNK), jnp.float32),
            pltpu.VMEM((_TPW,), jnp.float32),
            pltpu.VMEM((_NBLK, _GCHUNK), jnp.int32),
            pltpu.VMEM((_TPW, _C), jnp.float32),
            pltpu.VMEM((_C, _GCHUNK), jnp.float32),
            pltpu.VMEM((_L,), jnp.float32),
            pltpu.VMEM((_L,), jnp.int32),
            pltpu.SemaphoreType.DMA,
        ],
    )
    return f(P, idx3, vals3, output_weights)


def kernel(latents, output_task_indices, output_values, output_weights, W, b):
    latents_flat = latents.reshape(_B * _N, _D)
    P = _project(latents_flat, W, b.reshape(1, _C))
    idx3 = output_task_indices.reshape(_T // 128, 128, 2).transpose(0, 2, 1)
    vals3 = output_values.reshape(_T // 128, 128, _C).transpose(0, 2, 1)
    out3, loss_parts, pres = _sc_call(P, idx3, vals3, output_weights)
    out = out3.transpose(0, 2, 1).reshape(_T, _C)
    total = jnp.sum(loss_parts)
    nbatch = jnp.sum((jnp.sum(pres, axis=0) > 0).astype(jnp.float32))
    loss = total / (_T * _C) * nbatch / _B
    return out, loss


# speedup vs baseline: 5.0756x; 1.0288x over previous
import jax
import jax.numpy as jnp
from jax import lax
from jax.experimental import pallas as pl
from jax.experimental.pallas import tpu as pltpu
from jax.experimental.pallas import tpu_sc as plsc

_B, _N, _D, _C = 16, 4096, 1024, 8
_T = 32768

_NC, _NS, _L = 2, 16, 16
_NW = _NC * _NS
_TPW = _T // _NW
_GCHUNK = 128
_NBLK = _TPW // _GCHUNK
_GPB = _GCHUNK // _L


def _proj_body(l_ref, wt_ref, b_ref, p_ref):
    p_ref[...] = (
        lax.dot_general(l_ref[...], wt_ref[...], (((1,), (1,)), ((), ())),
                        preferred_element_type=jnp.float32)
        + b_ref[...]
    )


def _project(latents_flat, WT, b2):
    M = latents_flat.shape[0]
    BM = 2048
    return pl.pallas_call(
        _proj_body,
        grid=(M // BM,),
        in_specs=[
            pl.BlockSpec((BM, _D), lambda i: (i, 0)),
            pl.BlockSpec((_C, _D), lambda i: (0, 0)),
            pl.BlockSpec((1, _C), lambda i: (0, 0)),
        ],
        out_specs=pl.BlockSpec((BM, _C), lambda i: (i, 0)),
        out_shape=jax.ShapeDtypeStruct((M, _C), jnp.float32),
    )(latents_flat, WT, b2)


def _sc_body(p_hbm, idx3_hbm, vals3_hbm, w_hbm,
             out3_hbm, loss_hbm, pres_hbm,
             idx_v, vals_v, w_v, gidx_v, rows_v,
             out_v, loss_v, pres_v,
             sem_i, sem_v, sem_w, sem_g, sem_o):
    cid = lax.axis_index("c")
    sid = lax.axis_index("s")
    wid = sid * _NC + cid
    base = wid * _TPW
    t0 = wid * _NBLK

    h_idx = pltpu.async_copy(idx3_hbm.at[pl.ds(t0, _NBLK)], idx_v, sem_i)
    h_vals = pltpu.async_copy(vals3_hbm.at[pl.ds(t0, _NBLK)], vals_v, sem_v)
    h_w = pltpu.async_copy(w_hbm.at[pl.ds(base, _TPW)], w_v, sem_w)

    pres_v[...] = jnp.zeros((_L,), jnp.int32)
    ones = jnp.ones((_L,), jnp.int32)
    h_idx.wait()

    copies = []
    for tb in range(_NBLK):
        for g in range(_GPB):
            sl = pl.ds(g * _L, _L)
            vb = idx_v[tb, 0, sl]
            vn = idx_v[tb, 1, sl]
            gidx_v[tb, sl] = vb * _N + vn
            plsc.store_scatter(pres_v, [vb], ones)
        copies.append(pltpu.async_copy(
            p_hbm.at[gidx_v.at[tb]],
            rows_v.at[pl.ds(tb * _GCHUNK, _GCHUNK)],
            sem_g,
        ))
    h_vals.wait()
    h_w.wait()

    acc = jnp.zeros((_L,), jnp.float32)
    out_copies = [None, None]
    for tb in range(_NBLK):
        copies[tb].wait()
        buf = tb % 2
        if out_copies[buf] is not None:
            out_copies[buf].wait()
        for g in range(_GPB):
            ids = lax.iota(jnp.int32, _L) + (tb * _GCHUNK + g * _L)
            sl = pl.ds(g * _L, _L)
            s = jnp.zeros((_L,), jnp.float32)
            for c in range(_C):
                cc = jnp.full((_L,), c, jnp.int32)
                rv = plsc.load_gather(rows_v, [ids, cc])
                out_v[buf, c, sl] = rv
                d = rv - vals_v[tb, c, sl]
                s = s + d * d
            acc = acc + w_v[pl.ds(tb * _GCHUNK + g * _L, _L)] * s
        out_copies[buf] = pltpu.async_copy(
            out_v.at[buf], out3_hbm.at[t0 + tb], sem_o)

    loss_v[...] = acc
    pltpu.sync_copy(loss_v, loss_hbm.at[wid])
    pltpu.sync_copy(pres_v, pres_hbm.at[wid])
    out_copies[0].wait()
    out_copies[1].wait()


def _sc_call(P, idx3, vals3, output_weights):
    mesh = plsc.VectorSubcoreMesh(
        core_axis_name="c", subcore_axis_name="s",
        num_cores=_NC, num_subcores=_NS,
    )
    f = pl.kernel(
        _sc_body,
        out_type=[
            jax.ShapeDtypeStruct((_T // _GCHUNK, _C, _GCHUNK), jnp.float32),
            jax.ShapeDtypeStruct((_NW, _L), jnp.float32),
            jax.ShapeDtypeStruct((_NW, _L), jnp.int32),
        ],
        mesh=mesh,
        compiler_params=pltpu.CompilerParams(
            needs_layout_passes=False, use_tc_tiling_on_sc=False),
        scratch_types=[
            pltpu.VMEM((_NBLK, 2, _GCHUNK), jnp.int32),
            pltpu.VMEM((_NBLK, _C, _GCHUNK), jnp.float32),
            pltpu.VMEM((_TPW,), jnp.float32),
            pltpu.VMEM((_NBLK, _GCHUNK), jnp.int32),
            pltpu.VMEM((_TPW, _C), jnp.float32),
            pltpu.VMEM((2, _C, _GCHUNK), jnp.float32),
            pltpu.VMEM((_L,), jnp.float32),
            pltpu.VMEM((_L,), jnp.int32),
            pltpu.SemaphoreType.DMA,
            pltpu.SemaphoreType.DMA,
            pltpu.SemaphoreType.DMA,
            pltpu.SemaphoreType.DMA,
            pltpu.SemaphoreType.DMA,
        ],
    )
    return f(P, idx3, vals3, output_weights)


def kernel(latents, output_task_indices, output_values, output_weights, W, b):
    latents_flat = latents.reshape(_B * _N, _D)
    P = _project(latents_flat, W.T, b.reshape(1, _C))
    idx3 = output_task_indices.reshape(_T // 128, 128, 2).transpose(0, 2, 1)
    vals3 = output_values.reshape(_T // 128, 128, _C).transpose(0, 2, 1)
    out3, loss_parts, pres = _sc_call(P, idx3, vals3, output_weights)
    out = out3.transpose(0, 2, 1).reshape(_T, _C)
    total = jnp.sum(loss_parts)
    nbatch = jnp.sum((jnp.sum(pres, axis=0) > 0).astype(jnp.float32))
    loss = total / (_T * _C) * nbatch / _B
    return out, loss
